# baseline (device time: 144879 ns/iter reference)
import numpy as np

import jax
import jax.numpy as jnp
from jax import lax
from jax.experimental import pallas as pl
from jax.experimental.pallas import tpu as pltpu

N_DEV = 4


import os

_ROLL_MAX_J = int(os.environ.get("SORT_ROLL_MAX_J", "8"))
_NO_COMM = os.environ.get("SORT_NO_COMM", "0") == "1"


def _stage_roll(x, j, K, desc_traced):
    n, C = x.shape
    i = lax.broadcasted_iota(jnp.int32, (n, 1), 0)
    bitj = (i >> (j.bit_length() - 1)) % 2 == 1
    up = jnp.roll(x, -j, axis=0)
    dn = jnp.roll(x, j, axis=0)
    partner = jnp.where(bitj, dn, up)
    lo = jnp.minimum(x, partner)
    hi = jnp.maximum(x, partner)
    keep_hi = bitj
    if K is not None and K < n:
        keep_hi = keep_hi ^ ((i >> (K.bit_length() - 1)) % 2 == 1)
    if desc_traced is not None:
        keep_hi = keep_hi ^ desc_traced
    return jnp.where(keep_hi, hi, lo)


def _cmp_ex_const(x, K, j):
    n, C = x.shape
    if j < _ROLL_MAX_J:
        return _stage_roll(x, j, K, None)
    nb = n // (2 * j)
    a = x.reshape(nb, 2, j, C)
    lo = jnp.minimum(a[:, 0], a[:, 1])
    hi = jnp.maximum(a[:, 0], a[:, 1])
    if (nb - 1) * 2 * j < K:
        first, second = lo, hi
    else:
        shift = (K // (2 * j)).bit_length() - 1
        b = lax.broadcasted_iota(jnp.int32, (nb, 1, 1), 0)
        m = (b >> shift) % 2 == 1
        first = jnp.where(m, hi, lo)
        second = jnp.where(m, lo, hi)
    return jnp.stack([first, second], axis=1).reshape(n, C)


def _cmp_ex_dir(x, j, desc):
    n, C = x.shape
    if j < _ROLL_MAX_J:
        return _stage_roll(x, j, None, desc)
    nb = n // (2 * j)
    a = x.reshape(nb, 2, j, C)
    lo = jnp.minimum(a[:, 0], a[:, 1])
    hi = jnp.maximum(a[:, 0], a[:, 1])
    first = jnp.where(desc, hi, lo)
    second = jnp.where(desc, lo, hi)
    return jnp.stack([first, second], axis=1).reshape(n, C)


def _merge_dir(x, desc):
    j = x.shape[0] // 2
    while j >= 1:
        x = _cmp_ex_dir(x, j, desc)
        j //= 2
    return x


def _local_sort_levels(x):
    n = x.shape[0]
    K = 2
    while K <= n // 2:
        j = K // 2
        while j >= 1:
            x = _cmp_ex_const(x, K, j)
            j //= 2
        K *= 2
    return x


def _merge_asc(x):
    j = x.shape[0] // 2
    while j >= 1:
        x = _cmp_ex_const(x, 2 * x.shape[0], j)
        j //= 2
    return x


_N_CHUNKS = 4


def kernel(x):
    m_per, n_cols = x.shape
    cw = n_cols // _N_CHUNKS

    def body(x_ref, out_ref, work_ref, comm_ref, send_sems, recv_sems):
        my = lax.axis_index("i")
        p1 = my ^ 1
        p2 = my ^ 2

        barrier_sem = pltpu.get_barrier_semaphore()
        for nbr in (p1, p2):
            pl.semaphore_signal(
                barrier_sem, inc=1,
                device_id=(nbr,), device_id_type=pl.DeviceIdType.MESH,
            )
        pl.semaphore_wait(barrier_sem, 2)

        def start_ex(slot, partner, c, val):
            if _NO_COMM:
                return None
            work_ref[c, ...] = val
            rdma = pltpu.make_async_remote_copy(
                src_ref=work_ref.at[c],
                dst_ref=comm_ref.at[slot],
                send_sem=send_sems.at[slot],
                recv_sem=recv_sems.at[slot],
                device_id=(partner,),
                device_id_type=pl.DeviceIdType.MESH,
            )
            rdma.start()
            return rdma

        def finish_ex(rdma, slot, keep_max, val):
            if _NO_COMM:
                return val
            rdma.wait()
            r = comm_ref[slot]
            return jnp.where(keep_max, jnp.maximum(val, r), jnp.minimum(val, r))

        parity = my % 2 == 1
        upper = my >= 2
        km0 = (my == 1) | (my == 2)

        xv = _local_sort_levels(x_ref[...].astype(jnp.bfloat16))

        nc = _N_CHUNKS
        xc = [xv[:, c * cw:(c + 1) * cw] for c in range(nc)]
        r0, r1, r2 = [None] * nc, [None] * nc, [None] * nc

        for c in range(nc):
            xc[c] = _merge_dir(xc[c], parity)
            r0[c] = start_ex(c, p1, c, xc[c])

        for c in range(nc):
            xc[c] = finish_ex(r0[c], c, km0, xc[c])
            xc[c] = _merge_dir(xc[c], upper)
            r1[c] = start_ex(nc + c, p2, c, xc[c])

        for c in range(nc):
            xc[c] = finish_ex(r1[c], nc + c, upper, xc[c])
            r2[c] = start_ex(2 * nc + c, p1, c, xc[c])
            if c > 0:
                d = c - 1
                xc[d] = finish_ex(r2[d], 2 * nc + d, parity, xc[d])
                xc[d] = _merge_asc(xc[d])
                out_ref[:, d * cw:(d + 1) * cw] = xc[d].astype(jnp.float32)
        d = nc - 1
        xc[d] = finish_ex(r2[d], 2 * nc + d, parity, xc[d])
        xc[d] = _merge_asc(xc[d])
        out_ref[:, d * cw:(d + 1) * cw] = xc[d].astype(jnp.float32)

    return pl.pallas_call(
        body,
        out_shape=jax.ShapeDtypeStruct((m_per, n_cols), jnp.float32),
        in_specs=[pl.BlockSpec(memory_space=pltpu.VMEM)],
        out_specs=pl.BlockSpec(memory_space=pltpu.VMEM),
        scratch_shapes=[
            pltpu.VMEM((_N_CHUNKS, m_per, cw), jnp.bfloat16),
            pltpu.VMEM((3 * _N_CHUNKS, m_per, cw), jnp.bfloat16),
            pltpu.SemaphoreType.DMA((3 * _N_CHUNKS,)),
            pltpu.SemaphoreType.DMA((3 * _N_CHUNKS,)),
        ],
        compiler_params=pltpu.CompilerParams(
            collective_id=0, vmem_limit_bytes=100 * 1024 * 1024
        ),
    )(x)


# device time: 140724 ns/iter; 1.0295x vs baseline; 1.0295x over previous
import numpy as np

import jax
import jax.numpy as jnp
from jax import lax
from jax.experimental import pallas as pl
from jax.experimental.pallas import tpu as pltpu

N_DEV = 4


import os

_ROLL_MAX_J = int(os.environ.get("SORT_ROLL_MAX_J", "8"))
_NO_COMM = os.environ.get("SORT_NO_COMM", "0") == "1"
_COMM_ONLY = os.environ.get("SORT_COMM_ONLY", "0") == "1"


def _stage_roll(x, j, K, desc_traced):
    n, C = x.shape
    i = lax.broadcasted_iota(jnp.int32, (n, 1), 0)
    bitj = (i >> (j.bit_length() - 1)) % 2 == 1
    up = jnp.roll(x, -j, axis=0)
    dn = jnp.roll(x, j, axis=0)
    partner = jnp.where(bitj, dn, up)
    lo = jnp.minimum(x, partner)
    hi = jnp.maximum(x, partner)
    keep_hi = bitj
    if K is not None and K < n:
        keep_hi = keep_hi ^ ((i >> (K.bit_length() - 1)) % 2 == 1)
    if desc_traced is not None:
        keep_hi = keep_hi ^ desc_traced
    return jnp.where(keep_hi, hi, lo)


def _cmp_ex_const(x, K, j):
    n, C = x.shape
    if j < _ROLL_MAX_J:
        return _stage_roll(x, j, K, None)
    nb = n // (2 * j)
    a = x.reshape(nb, 2, j, C)
    lo = jnp.minimum(a[:, 0], a[:, 1])
    hi = jnp.maximum(a[:, 0], a[:, 1])
    if (nb - 1) * 2 * j < K:
        first, second = lo, hi
    else:
        shift = (K // (2 * j)).bit_length() - 1
        b = lax.broadcasted_iota(jnp.int32, (nb, 1, 1), 0)
        m = (b >> shift) % 2 == 1
        first = jnp.where(m, hi, lo)
        second = jnp.where(m, lo, hi)
    return jnp.stack([first, second], axis=1).reshape(n, C)


def _cmp_ex_dir(x, j, desc):
    n, C = x.shape
    if j < _ROLL_MAX_J:
        return _stage_roll(x, j, None, desc)
    nb = n // (2 * j)
    a = x.reshape(nb, 2, j, C)
    lo = jnp.minimum(a[:, 0], a[:, 1])
    hi = jnp.maximum(a[:, 0], a[:, 1])
    first = jnp.where(desc, hi, lo)
    second = jnp.where(desc, lo, hi)
    return jnp.stack([first, second], axis=1).reshape(n, C)


def _merge_dir(x, desc):
    if _COMM_ONLY:
        return x
    j = x.shape[0] // 2
    while j >= 1:
        x = _cmp_ex_dir(x, j, desc)
        j //= 2
    return x


def _local_sort_levels(x):
    if _COMM_ONLY:
        return x
    n = x.shape[0]
    K = 2
    while K <= n // 2:
        j = K // 2
        while j >= 1:
            x = _cmp_ex_const(x, K, j)
            j //= 2
        K *= 2
    return x


def _merge_asc(x):
    if _COMM_ONLY:
        return x
    j = x.shape[0] // 2
    while j >= 1:
        x = _cmp_ex_const(x, 2 * x.shape[0], j)
        j //= 2
    return x


_N_CHUNKS = 4


def kernel(x):
    m_per, n_cols = x.shape
    cw = n_cols // _N_CHUNKS

    def body(x_ref, out_ref, work_ref, comm_ref, send_sems, recv_sems):
        my = lax.axis_index("i")
        p1 = my ^ 1
        p2 = my ^ 2

        barrier_sem = pltpu.get_barrier_semaphore()
        for nbr in (p1, p2):
            pl.semaphore_signal(
                barrier_sem, inc=1,
                device_id=(nbr,), device_id_type=pl.DeviceIdType.MESH,
            )
        pl.semaphore_wait(barrier_sem, 2)

        def start_ex(slot, partner, c, val):
            if _NO_COMM:
                return None
            work_ref[c, ...] = val
            rdma = pltpu.make_async_remote_copy(
                src_ref=work_ref.at[c],
                dst_ref=comm_ref.at[slot],
                send_sem=send_sems.at[slot],
                recv_sem=recv_sems.at[slot],
                device_id=(partner,),
                device_id_type=pl.DeviceIdType.MESH,
            )
            rdma.start()
            return rdma

        def finish_ex(rdma, slot, keep_max, val):
            if _NO_COMM:
                return val
            rdma.wait()
            r = comm_ref[slot]
            return jnp.where(keep_max, jnp.maximum(val, r), jnp.minimum(val, r))

        parity = my % 2 == 1
        upper = my >= 2
        km0 = (my == 1) | (my == 2)

        nc = _N_CHUNKS
        xc = [None] * nc
        r0, r1, r2 = [None] * nc, [None] * nc, [None] * nc

        half = n_cols // 2
        xh = x_ref[:, :half].astype(jnp.bfloat16)
        xh = _local_sort_levels(xh)
        for c in range(nc // 2):
            xc[c] = _merge_dir(xh[:, c * cw:(c + 1) * cw], parity)
            r0[c] = start_ex(c, p1, c, xc[c])
        xh = x_ref[:, half:].astype(jnp.bfloat16)
        xh = _local_sort_levels(xh)
        for c in range(nc // 2, nc):
            xc[c] = _merge_dir(xh[:, (c - nc // 2) * cw:(c - nc // 2 + 1) * cw], parity)
            r0[c] = start_ex(c, p1, c, xc[c])

        for c in range(nc):
            xc[c] = finish_ex(r0[c], c, km0, xc[c])
            xc[c] = _merge_dir(xc[c], upper)
            r1[c] = start_ex(nc + c, p2, c, xc[c])

        for c in range(nc):
            xc[c] = finish_ex(r1[c], nc + c, upper, xc[c])
            r2[c] = start_ex(2 * nc + c, p1, c, xc[c])
            if c > 0:
                d = c - 1
                xc[d] = finish_ex(r2[d], 2 * nc + d, parity, xc[d])
                xc[d] = _merge_asc(xc[d])
                out_ref[:, d * cw:(d + 1) * cw] = xc[d].astype(jnp.float32)
        d = nc - 1
        xc[d] = finish_ex(r2[d], 2 * nc + d, parity, xc[d])
        xc[d] = _merge_asc(xc[d])
        out_ref[:, d * cw:(d + 1) * cw] = xc[d].astype(jnp.float32)

    return pl.pallas_call(
        body,
        out_shape=jax.ShapeDtypeStruct((m_per, n_cols), jnp.float32),
        in_specs=[pl.BlockSpec(memory_space=pltpu.VMEM)],
        out_specs=pl.BlockSpec(memory_space=pltpu.VMEM),
        scratch_shapes=[
            pltpu.VMEM((_N_CHUNKS, m_per, cw), jnp.bfloat16),
            pltpu.VMEM((3 * _N_CHUNKS, m_per, cw), jnp.bfloat16),
            pltpu.SemaphoreType.DMA((3 * _N_CHUNKS,)),
            pltpu.SemaphoreType.DMA((3 * _N_CHUNKS,)),
        ],
        compiler_params=pltpu.CompilerParams(
            collective_id=0, vmem_limit_bytes=100 * 1024 * 1024
        ),
    )(x)
